# baseline (speedup 1.0000x reference)
import jax
import jax.numpy as jnp
import numpy as np
from jax.experimental import pallas as pl
from jax.experimental.pallas import tpu as pltpu

N = 65536
D = 512
K = 16384
LANES = 128
SUBL = N // LANES
R = 32
BN = R * LANES

_MININT = np.int32(-(2 ** 31))


def _sortable_key(f32val):
    b = jax.lax.bitcast_convert_type(f32val, jnp.int32)
    return jnp.where(b < 0, jnp.bitwise_xor(jnp.bitwise_not(b), _MININT), b)


def _mask_body(attn2d_ref, x_ref, o_ref, keep_ref, sel_ref):
    step = pl.program_id(0)

    @pl.when(step == 0)
    def _select():
        key = _sortable_key(attn2d_ref[...])

        def cnt_ge(c):
            return jnp.sum((key >= c).astype(jnp.int32))

        prefix = _MININT
        c_ge = np.int32(N)
        for b in range(31, 0, -2):
            lo = np.int32(1 << (b - 1))
            if b == 31:
                c_hi = np.int32(0)
                c_lo0 = np.int32(_MININT + lo)
            else:
                c_hi = prefix + np.int32(1 << b)
                c_lo0 = prefix + lo
            n_hi = cnt_ge(c_hi)
            n_lo0 = cnt_ge(c_lo0)
            n_lo1 = cnt_ge(c_hi + lo)
            take_hi = n_hi >= K
            prefix = jnp.where(take_hi, c_hi, prefix)
            c_ge = jnp.where(take_hi, n_hi, c_ge)
            n_next = jnp.where(take_hi, n_lo1, n_lo0)
            take_lo = n_next >= K
            prefix = jnp.where(take_lo, prefix + lo, prefix)
            c_ge = jnp.where(take_lo, n_next, c_ge)
        t_key = prefix

        row = jax.lax.broadcasted_iota(jnp.int32, (SUBL, LANES), 0)
        col = jax.lax.broadcasted_iota(jnp.int32, (SUBL, LANES), 1)
        idx = row * LANES + col
        tied = key == t_key
        sel_ref[0] = np.int32(N - 1)

        @pl.when(c_ge > K)
        def _resolve_ties():
            c_gt = jnp.sum((key > t_key).astype(jnp.int32))
            budget = np.int32(K) - c_gt

            def cnt_le(bound):
                return jnp.sum((tied & (idx <= bound)).astype(jnp.int32))

            ipfx = np.int32(0)
            for b in range(15, 0, -2):
                hi = np.int32(1 << b)
                lo = np.int32(1 << (b - 1))
                c1 = cnt_le(ipfx + hi - np.int32(1))
                c2a = cnt_le(ipfx + lo - np.int32(1))
                c2b = cnt_le(ipfx + hi + lo - np.int32(1))
                keep_hi0 = c1 >= budget
                ipfx = jnp.where(keep_hi0, ipfx, ipfx + hi)
                c2 = jnp.where(keep_hi0, c2a, c2b)
                ipfx = jnp.where(c2 >= budget, ipfx, ipfx + lo)
            sel_ref[0] = ipfx

        i_star = sel_ref[0]
        masked = (key > t_key) | (tied & (idx <= i_star))
        keep_ref[...] = jnp.where(masked, np.float32(0.0), np.float32(1.0))

    keep = keep_ref[pl.ds(step * R, R), :]
    o_ref[...] = x_ref[...] * keep[:, :, None]


@jax.jit
def kernel(x, attn):
    x3 = x.reshape(SUBL, LANES, D)
    attn2d = attn.reshape(SUBL, LANES)

    out = pl.pallas_call(
        _mask_body,
        grid=(SUBL // R,),
        in_specs=[
            pl.BlockSpec((SUBL, LANES), lambda i: (0, 0)),
            pl.BlockSpec((R, LANES, D), lambda i: (i, 0, 0)),
        ],
        out_specs=pl.BlockSpec((R, LANES, D), lambda i: (i, 0, 0)),
        out_shape=jax.ShapeDtypeStruct((SUBL, LANES, D), jnp.float32),
        scratch_shapes=[
            pltpu.VMEM((SUBL, LANES), jnp.float32),
            pltpu.SMEM((1,), jnp.int32),
        ],
        compiler_params=pltpu.CompilerParams(
            dimension_semantics=("arbitrary",),
        ),
    )(attn2d, x3)
    return out.reshape(1, N, D)
